# baseline (speedup 1.0000x reference)
import jax
import jax.numpy as jnp
from jax.experimental import pallas as pl
from jax.experimental.pallas import tpu as pltpu

N_NODES = 1000000
MEM_DIM = 32
BATCH = 16384
COPY_ROWS = 8000


def _copy_body(src_ref, dst_ref):
    dst_ref[...] = src_ref[...]


def _sg_body(idx_ref, val_ref, lu_ref, mem_any, out_mem_ref, out_lu_ref):
    del idx_ref, mem_any
    out_mem_ref[...] = val_ref[...]
    out_lu_ref[...] = lu_ref[...]


def kernel(memory, last_update, node_idxs, values):
    mem3 = memory.reshape(N_NODES, 1, MEM_DIM)
    vals3 = values.reshape(BATCH, 1, MEM_DIM)
    lu3 = last_update.reshape(N_NODES, 1, 1)

    copied = pl.pallas_call(
        _copy_body,
        grid=(N_NODES // COPY_ROWS,),
        in_specs=[pl.BlockSpec((COPY_ROWS, 1, MEM_DIM), lambda i: (i, 0, 0))],
        out_specs=pl.BlockSpec((COPY_ROWS, 1, MEM_DIM), lambda i: (i, 0, 0)),
        out_shape=jax.ShapeDtypeStruct((N_NODES, 1, MEM_DIM), jnp.float32),
    )(mem3)

    grid_spec = pltpu.PrefetchScalarGridSpec(
        num_scalar_prefetch=1,
        grid=(BATCH,),
        in_specs=[
            pl.BlockSpec((1, 1, MEM_DIM), lambda i, idx: (i, 0, 0)),
            pl.BlockSpec((1, 1, 1), lambda i, idx: (idx[i], 0, 0)),
            pl.BlockSpec(memory_space=pl.ANY),
        ],
        out_specs=[
            pl.BlockSpec((1, 1, MEM_DIM), lambda i, idx: (idx[i], 0, 0)),
            pl.BlockSpec((1, 1, 1), lambda i, idx: (i, 0, 0)),
        ],
    )
    new_mem3, out_lu3 = pl.pallas_call(
        _sg_body,
        grid_spec=grid_spec,
        out_shape=[
            jax.ShapeDtypeStruct((N_NODES, 1, MEM_DIM), jnp.float32),
            jax.ShapeDtypeStruct((BATCH, 1, 1), jnp.float32),
        ],
        input_output_aliases={3: 0},
    )(node_idxs, vals3, lu3, copied)

    return new_mem3.reshape(N_NODES, MEM_DIM), out_lu3.reshape(BATCH)


# speedup vs baseline: 10.7265x; 10.7265x over previous
import jax
import jax.numpy as jnp
from jax import lax
from jax.experimental import pallas as pl
from jax.experimental.pallas import tpu as pltpu
from jax.experimental.pallas import tpu_sc as plsc

N_NODES = 1000000
MEM_DIM = 32
BATCH = 16384
L = 16
NC, NS = 2, 16
NW = NC * NS
BPW = BATCH // NW
NCHUNK = BATCH // L

BUCKET_ROWS = 20000
NBLOCKS = N_NODES // BUCKET_ROWS
NB = 64
PACK = BATCH + NB * (L - 1) + L
POS_BITS = 14


def _sc1_body(lu_hbm, idx_hbm, cnt_hbm, out_lu_hbm, idx_v, lu_v, cnt_v, sem):
    wid = lax.axis_index("s") * NC + lax.axis_index("c")
    bbase = wid * BPW
    iota = lax.iota(jnp.int32, L)

    pltpu.sync_copy(idx_hbm, idx_v)

    my_idx = idx_v.at[pl.ds(bbase, BPW)]
    pltpu.async_copy(lu_hbm.at[my_idx], lu_v, sem).wait()
    pltpu.sync_copy(lu_v, out_lu_hbm.at[pl.ds(bbase, BPW)])

    lo0 = wid * BUCKET_ROWS
    hi0 = lo0 + BUCKET_ROWS
    lo1 = (wid + NW) * BUCKET_ROWS
    hi1 = lo1 + BUCKET_ROWS

    def scan(c, carry):
        c0, c1 = carry
        v = idx_v[pl.ds(c * L, L)]
        c0 = c0 + jnp.sum(((v >= lo0) & (v < hi0)).astype(jnp.int32))
        c1 = c1 + jnp.sum(((v >= lo1) & (v < hi1)).astype(jnp.int32))
        return c0, c1

    c0, c1 = lax.fori_loop(0, NCHUNK, scan, (jnp.int32(0), jnp.int32(0)))
    cnt_v[pl.ds(0, L)] = jnp.where(iota == 0, c0, jnp.where(iota == 1, c1, 0))
    pltpu.sync_copy(cnt_v, cnt_hbm.at[pl.ds(wid * L, L)])


_sc1 = pl.kernel(
    _sc1_body,
    out_type=[
        jax.ShapeDtypeStruct((NW * L,), jnp.int32),
        jax.ShapeDtypeStruct((BATCH,), jnp.float32),
    ],
    mesh=plsc.VectorSubcoreMesh(core_axis_name="c", subcore_axis_name="s",
                                num_cores=NC, num_subcores=NS),
    compiler_params=pltpu.CompilerParams(needs_layout_passes=False),
    scratch_types=[
        pltpu.VMEM((BATCH,), jnp.int32),
        pltpu.VMEM((BPW,), jnp.float32),
        pltpu.VMEM((L,), jnp.int32),
        pltpu.SemaphoreType.DMA,
    ],
)


def _sc2_body(idx_hbm, off_hbm, pk_hbm, idx_v, off_v, comp, sem):
    wid = lax.axis_index("s") * NC + lax.axis_index("c")
    iota = lax.iota(jnp.int32, L)

    pltpu.sync_copy(idx_hbm, idx_v)
    pltpu.sync_copy(off_hbm, off_v)

    def do_bucket(b):
        lo = b * BUCKET_ROWS
        hi = lo + BUCKET_ROWS
        off = jnp.max(plsc.load_gather(off_v, [jnp.full((L,), b, jnp.int32)]))
        off = pl.multiple_of(off, L)

        def scan(c, n):
            v = idx_v[pl.ds(c * L, L)]
            m = (v >= lo) & (v < hi)
            mi = m.astype(jnp.int32)
            pos = c * L + iota
            packed = ((v - lo) << POS_BITS) | pos
            o = jnp.maximum(n + plsc.cumsum(mi) - 1, 0)
            plsc.store_scatter(comp, [o], packed, mask=m)
            return n + jnp.sum(mi)

        n = lax.fori_loop(0, NCHUNK, scan, jnp.int32(0))

        def wr(k, carry):
            pltpu.sync_copy(comp.at[pl.ds(k * L, L)],
                            pk_hbm.at[pl.ds(off + k * L, L)])
            return carry

        lax.fori_loop(0, (n + L - 1) // L, wr, jnp.int32(0))

    do_bucket(wid)
    do_bucket(wid + NW)


_sc2 = pl.kernel(
    _sc2_body,
    out_type=jax.ShapeDtypeStruct((PACK,), jnp.int32),
    mesh=plsc.VectorSubcoreMesh(core_axis_name="c", subcore_axis_name="s",
                                num_cores=NC, num_subcores=NS),
    compiler_params=pltpu.CompilerParams(needs_layout_passes=False),
    scratch_types=[
        pltpu.VMEM((BATCH,), jnp.int32),
        pltpu.VMEM((NB + 8,), jnp.int32),
        pltpu.VMEM((BATCH + L,), jnp.int32),
        pltpu.SemaphoreType.DMA,
    ],
)


def _tc_body(pk_ref, off_ref, cnt_ref, src_ref, val_ref, out_ref):
    i = pl.program_id(0)
    out_ref[...] = src_ref[...]
    start = off_ref[i]

    def apply(j, carry):
        p = pk_ref[j]
        local = lax.shift_right_logical(p, POS_BITS)
        pos = p & ((1 << POS_BITS) - 1)
        out_ref[pl.ds(local, 1), :] = val_ref[pl.ds(pos, 1), :]
        return carry

    lax.fori_loop(start, start + cnt_ref[i], apply, jnp.int32(0))


def kernel(memory, last_update, node_idxs, values):
    counts_raw, out_lu = _sc1(last_update, node_idxs)

    cnts = counts_raw.reshape(NW, L)
    counts64 = jnp.concatenate([cnts[:, 0], cnts[:, 1]])
    padded = (counts64 + (L - 1)) // L * L
    offsets = jnp.concatenate(
        [jnp.zeros((1,), jnp.int32), jnp.cumsum(padded, dtype=jnp.int32)]
    )
    off_in = jnp.pad(offsets, (0, NB + 8 - offsets.shape[0]))

    packed = _sc2(node_idxs, off_in)

    grid_spec = pltpu.PrefetchScalarGridSpec(
        num_scalar_prefetch=3,
        grid=(NBLOCKS,),
        in_specs=[
            pl.BlockSpec((BUCKET_ROWS, MEM_DIM), lambda i, pk, of, ct: (i, 0)),
            pl.BlockSpec((BATCH, MEM_DIM), lambda i, pk, of, ct: (0, 0)),
        ],
        out_specs=pl.BlockSpec((BUCKET_ROWS, MEM_DIM),
                               lambda i, pk, of, ct: (i, 0)),
    )
    new_memory = pl.pallas_call(
        _tc_body,
        grid_spec=grid_spec,
        out_shape=jax.ShapeDtypeStruct((N_NODES, MEM_DIM), jnp.float32),
    )(packed, offsets, counts64, memory, values)

    return new_memory, out_lu
